# baseline (speedup 1.0000x reference)
import jax
import jax.numpy as jnp
import numpy as np
from jax import lax
from jax.experimental import pallas as pl
from jax.experimental.pallas import tpu as pltpu
from jax.experimental.pallas import tpu_sc as plsc

_NUM_TILES = 32
_SUB = 128
_NSUB = 4
_FEAT = 128

def _perm(v, perm):
    return jnp.take_along_axis(v, perm, axis=0, mode="promise_in_bounds")


def _subchunk(xb, cb, p_v, total, lanes):
    last = lanes == 15

    @plsc.parallel_loop(0, _SUB, 2, unroll=4, carry=total)
    def rloop(i, tot):
        for rr in range(2):
            r = i + rr
            p = None
            for j in range(_FEAT // 16):
                xv = xb[r, pl.ds(j * 16, 16)]
                cv = cb[r, pl.ds(j * 16, 16)]
                d = xv - cv
                dd = d * d
                p = dd if p is None else p + dd
            cs = jnp.cumsum(p)
            tot = tot + jnp.where(last, jnp.clip(cs, 1e-12, 1e12), 0.0)
        return tot

    return rloop


def _body(x_hbm, labels_hbm, centers_hbm, out_hbm,
          lbl_v, x0, x1, c0, c1, p_v, res_v, semx, semc):
    wid = lax.axis_index("s") * 2 + lax.axis_index("c")
    base = wid * (_NSUB * _SUB)
    lanes = lax.iota(jnp.int32, 16)

    pltpu.sync_copy(labels_hbm.at[pl.ds(base, _NSUB * _SUB)], lbl_v)

    xbufs = (x0, x1)
    cbufs = (c0, c1)
    sxs = (semx.at[0], semx.at[1])
    scs = (semc.at[0], semc.at[1])

    def start(s):
        b = s % 2
        hc = pltpu.async_copy(
            centers_hbm.at[lbl_v.at[pl.ds(s * _SUB, _SUB)]], cbufs[b], scs[b])
        hx = pltpu.async_copy(
            x_hbm.at[pl.ds(base + s * _SUB, _SUB)], xbufs[b], sxs[b])
        return hc, hx

    pend = start(0)
    total = jnp.zeros((16,), jnp.float32)
    for s in range(_NSUB):
        hc, hx = pend
        if s + 1 < _NSUB:
            nxt = start(s + 1)
        hc.wait()
        hx.wait()
        b = s % 2
        total = _subchunk(xbufs[b], cbufs[b], p_v, total, lanes)
        if s + 1 < _NSUB:
            pend = nxt

    res_v[...] = total
    pltpu.sync_copy(res_v, out_hbm.at[wid])


@jax.jit
def kernel(x, labels, centers):
    batch = x.shape[0]
    labels = labels.astype(jnp.int32)

    k = pl.kernel(
        _body,
        out_type=jax.ShapeDtypeStruct((_NUM_TILES, 16), jnp.float32),
        mesh=plsc.VectorSubcoreMesh(core_axis_name="c", subcore_axis_name="s"),
        compiler_params=pltpu.CompilerParams(needs_layout_passes=False),
        scratch_types=[
            pltpu.VMEM((_NSUB * _SUB,), jnp.int32),
            pltpu.VMEM((_SUB, _FEAT), jnp.float32),
            pltpu.VMEM((_SUB, _FEAT), jnp.float32),
            pltpu.VMEM((_SUB, _FEAT), jnp.float32),
            pltpu.VMEM((_SUB, _FEAT), jnp.float32),
            pltpu.VMEM((_SUB * 16,), jnp.float32),
            pltpu.VMEM((16,), jnp.float32),
            pltpu.SemaphoreType.DMA((2,)),
            pltpu.SemaphoreType.DMA((2,)),
        ],
    )
    partials = k(x, labels, centers)
    return jnp.sum(partials) / batch


# speedup vs baseline: 1.1335x; 1.1335x over previous
import jax
import jax.numpy as jnp
from jax import lax
from jax.experimental import pallas as pl
from jax.experimental.pallas import tpu as pltpu
from jax.experimental.pallas import tpu_sc as plsc

_NUM_TILES = 32
_SUB = 128
_NSUB = 4
_FEAT = 128

def _subchunk(xb, cb, total):
    @plsc.parallel_loop(0, _SUB, 2, unroll=2, carry=total)
    def rloop(i, tot):
        for rr in range(2):
            r = i + rr
            p = None
            for j in range(_FEAT // 16):
                xv = xb[r, pl.ds(j * 16, 16)]
                cv = cb[r, pl.ds(j * 16, 16)]
                d = xv - cv
                dd = d * d
                p = dd if p is None else p + dd
            cs = jnp.cumsum(p)
            tot = tot + jnp.clip(cs, 1e-12, 1e12)
        return tot

    return rloop


def _body(x_hbm, labels_hbm, centers_hbm, out_hbm,
          lbl_v, x0, x1, c0, c1, res_v, semx, semc):
    wid = lax.axis_index("s") * 2 + lax.axis_index("c")
    base = wid * (_NSUB * _SUB)
    lanes = lax.iota(jnp.int32, 16)

    pltpu.sync_copy(labels_hbm.at[pl.ds(base, _NSUB * _SUB)], lbl_v)

    xbufs = (x0, x1)
    cbufs = (c0, c1)
    sxs = (semx.at[0], semx.at[1])
    scs = (semc.at[0], semc.at[1])

    def start(s):
        b = s % 2
        hc = pltpu.async_copy(
            centers_hbm.at[lbl_v.at[pl.ds(s * _SUB, _SUB)]], cbufs[b], scs[b])
        hx = pltpu.async_copy(
            x_hbm.at[pl.ds(base + s * _SUB, _SUB)], xbufs[b], sxs[b])
        return hc, hx

    pend = start(0)
    total = jnp.zeros((16,), jnp.float32)
    for s in range(_NSUB):
        hc, hx = pend
        if s + 1 < _NSUB:
            nxt = start(s + 1)
        hc.wait()
        hx.wait()
        b = s % 2
        total = _subchunk(xbufs[b], cbufs[b], total)
        if s + 1 < _NSUB:
            pend = nxt

    res_v[...] = jnp.where(lanes == 15, total, 0.0)
    pltpu.sync_copy(res_v, out_hbm.at[wid])


@jax.jit
def kernel(x, labels, centers):
    batch = x.shape[0]
    labels = labels.astype(jnp.int32)

    k = pl.kernel(
        _body,
        out_type=jax.ShapeDtypeStruct((_NUM_TILES, 16), jnp.float32),
        mesh=plsc.VectorSubcoreMesh(core_axis_name="c", subcore_axis_name="s"),
        compiler_params=pltpu.CompilerParams(needs_layout_passes=False),
        scratch_types=[
            pltpu.VMEM((_NSUB * _SUB,), jnp.int32),
            pltpu.VMEM((_SUB, _FEAT), jnp.float32),
            pltpu.VMEM((_SUB, _FEAT), jnp.float32),
            pltpu.VMEM((_SUB, _FEAT), jnp.float32),
            pltpu.VMEM((_SUB, _FEAT), jnp.float32),
            pltpu.VMEM((16,), jnp.float32),
            pltpu.SemaphoreType.DMA((2,)),
            pltpu.SemaphoreType.DMA((2,)),
        ],
    )
    partials = k(x, labels, centers)
    return jnp.sum(partials) / batch
